# baseline (device time: 31666 ns/iter reference)
import jax
import jax.numpy as jnp
from jax import lax
from jax.experimental import pallas as pl
from jax.experimental.pallas import tpu as pltpu

N_DEV = 8


def kernel(x, w_mat, scale_x, scale_w):
    m_per, k = x.shape
    _, n = w_mat.shape
    n_per = n // N_DEV
    m_out = m_per * N_DEV

    scale = (scale_x[0] * scale_w[0]).reshape(1, 1).astype(jnp.float32)

    def body(
        x_ref,
        w32_ref,
        scale_ref,
        out_ref,
        x8_ref,
        send_ref,
        recv_ref,
        send_sems,
        recv_sems,
    ):
        my = lax.axis_index("i")
        d = pl.program_id(0)
        last = N_DEV - 1

        barrier_sem = pltpu.get_barrier_semaphore()

        @pl.when(d == 0)
        def _():
            for p in range(N_DEV):
                pl.semaphore_signal(
                    barrier_sem,
                    inc=1,
                    device_id=(p,),
                    device_id_type=pl.DeviceIdType.MESH,
                )
            pl.semaphore_wait(barrier_sem, N_DEV)
            x8_ref[...] = x_ref[...].astype(jnp.float8_e5m2)

        acc = jnp.dot(
            x8_ref[...],
            w32_ref[...].astype(jnp.float8_e5m2),
            preferred_element_type=jnp.float32,
        )
        yblk = jnp.maximum(acc * scale_ref[0, 0], 0.0)

        @pl.when(d < last)
        def _():
            s = d + 1
            send_ref[:, pl.ds(s * n_per, n_per)] = yblk.astype(jnp.bfloat16)
            pltpu.make_async_remote_copy(
                src_ref=send_ref.at[:, pl.ds(s * n_per, n_per)],
                dst_ref=recv_ref.at[:, pl.ds(s * n_per, n_per)],
                send_sem=send_sems.at[s],
                recv_sem=recv_sems.at[s],
                device_id=(lax.rem(my + s, N_DEV),),
                device_id_type=pl.DeviceIdType.MESH,
            ).start()

        def drain(s):
            src = lax.rem(my - s + N_DEV, N_DEV)
            desc = pltpu.make_async_remote_copy(
                src_ref=send_ref.at[:, pl.ds(s * n_per, n_per)],
                dst_ref=recv_ref.at[:, pl.ds(s * n_per, n_per)],
                send_sem=send_sems.at[s],
                recv_sem=recv_sems.at[s],
                device_id=(lax.rem(my + s, N_DEV),),
                device_id_type=pl.DeviceIdType.MESH,
            )
            desc.wait_recv()
            out_ref[pl.ds(src * m_per, m_per), :] = recv_ref[
                :, pl.ds(s * n_per, n_per)
            ].astype(jnp.float32)
            desc.wait_send()

        @pl.when(d >= 2)
        def _():
            drain(d - 1)

        @pl.when(d == last)
        def _():
            out_ref[pl.ds(my * m_per, m_per), :] = yblk
            drain(last)

    def w_index_map(d):
        my = lax.axis_index("i")
        return (0, lax.rem(my + d + 1, N_DEV))

    return pl.pallas_call(
        body,
        grid=(N_DEV,),
        out_shape=jax.ShapeDtypeStruct((m_out, n_per), jnp.float32),
        in_specs=[
            pl.BlockSpec((m_per, k), lambda d: (0, 0)),
            pl.BlockSpec((k, n_per), w_index_map),
            pl.BlockSpec(memory_space=pltpu.SMEM),
        ],
        out_specs=pl.BlockSpec((m_out, n_per), lambda d: (0, 0)),
        scratch_shapes=[
            pltpu.VMEM((m_per, k), jnp.float8_e5m2),
            pltpu.VMEM((m_per, n), jnp.bfloat16),
            pltpu.VMEM((m_per, n), jnp.bfloat16),
            pltpu.SemaphoreType.DMA((N_DEV,)),
            pltpu.SemaphoreType.DMA((N_DEV,)),
        ],
        compiler_params=pltpu.CompilerParams(
            collective_id=0,
            dimension_semantics=("arbitrary",),
        ),
    )(x, w_mat, scale)


# device time: 29180 ns/iter; 1.0852x vs baseline; 1.0852x over previous
import jax
import jax.numpy as jnp
from jax import lax
from jax.experimental import pallas as pl
from jax.experimental.pallas import tpu as pltpu

N_DEV = 8


def kernel(x, w_mat, scale_x, scale_w):
    m_per, k = x.shape
    _, n = w_mat.shape
    n_per = n // N_DEV
    m_out = m_per * N_DEV

    scale = (scale_x[0] * scale_w[0]).reshape(1, 1).astype(jnp.float32)

    def body(
        x_ref,
        w32_ref,
        scale_ref,
        out_ref,
        x8_ref,
        send_ref,
        recv_ref,
        send_sems,
        recv_sems,
    ):
        my = lax.axis_index("i")
        d = pl.program_id(0)
        last = N_DEV - 1

        barrier_sem = pltpu.get_barrier_semaphore()

        @pl.when(d == 0)
        def _():
            for p in range(N_DEV):
                pl.semaphore_signal(
                    barrier_sem,
                    inc=1,
                    device_id=(p,),
                    device_id_type=pl.DeviceIdType.MESH,
                )
            pl.semaphore_wait(barrier_sem, N_DEV)
            x8_ref[...] = x_ref[...].astype(jnp.float8_e5m2)

        acc = jnp.dot(
            x8_ref[...],
            w32_ref[...].astype(jnp.float8_e5m2),
            preferred_element_type=jnp.float32,
        )
        yblk = jnp.maximum(acc * scale_ref[0, 0], 0.0)

        @pl.when(d < last)
        def _():
            s = d + 1
            send_ref[:, pl.ds(s * n_per, n_per)] = yblk.astype(jnp.bfloat16)
            pltpu.make_async_remote_copy(
                src_ref=send_ref.at[:, pl.ds(s * n_per, n_per)],
                dst_ref=recv_ref.at[:, pl.ds(s * n_per, n_per)],
                send_sem=send_sems.at[s],
                recv_sem=recv_sems.at[s],
                device_id=(lax.rem(my + s, N_DEV),),
                device_id_type=pl.DeviceIdType.MESH,
            ).start()

        @pl.when(d == last)
        def _():
            out_ref[pl.ds(my * m_per, m_per), :] = yblk
            for s in range(1, N_DEV):
                src = lax.rem(my - s + N_DEV, N_DEV)
                desc = pltpu.make_async_remote_copy(
                    src_ref=send_ref.at[:, pl.ds(s * n_per, n_per)],
                    dst_ref=recv_ref.at[:, pl.ds(s * n_per, n_per)],
                    send_sem=send_sems.at[s],
                    recv_sem=recv_sems.at[s],
                    device_id=(lax.rem(my + s, N_DEV),),
                    device_id_type=pl.DeviceIdType.MESH,
                )
                desc.wait_recv()
                out_ref[pl.ds(src * m_per, m_per), :] = recv_ref[
                    :, s * n_per : (s + 1) * n_per
                ].astype(jnp.float32)
                desc.wait_send()

    def w_index_map(d):
        my = lax.axis_index("i")
        return (0, lax.rem(my + d + 1, N_DEV))

    return pl.pallas_call(
        body,
        grid=(N_DEV,),
        out_shape=jax.ShapeDtypeStruct((m_out, n_per), jnp.float32),
        in_specs=[
            pl.BlockSpec((m_per, k), lambda d: (0, 0)),
            pl.BlockSpec((k, n_per), w_index_map),
            pl.BlockSpec(memory_space=pltpu.SMEM),
        ],
        out_specs=pl.BlockSpec((m_out, n_per), lambda d: (0, 0)),
        scratch_shapes=[
            pltpu.VMEM((m_per, k), jnp.float8_e5m2),
            pltpu.VMEM((m_per, n), jnp.bfloat16),
            pltpu.VMEM((m_per, n), jnp.bfloat16),
            pltpu.SemaphoreType.DMA((N_DEV,)),
            pltpu.SemaphoreType.DMA((N_DEV,)),
        ],
        compiler_params=pltpu.CompilerParams(
            collective_id=0,
            dimension_semantics=("arbitrary",),
        ),
    )(x, w_mat, scale)
